# baseline (speedup 1.0000x reference)
import functools

import jax
import jax.numpy as jnp
from jax import lax
from jax.experimental import pallas as pl
from jax.experimental.pallas import tpu as pltpu
from jax.experimental.pallas import tpu_sc as plsc

N_NODES = 100000
N_EDGES = 3200000
D = 16
NC, NS = 2, 16
NW = NC * NS
EW = N_EDGES // NW
C = 400
NCH = EW // C
GROUPS = C // 16

_f32 = jnp.float32
_i32 = jnp.int32


def _sc_edge_dots(zs, src, dst):
    mesh = plsc.VectorSubcoreMesh(core_axis_name="c", subcore_axis_name="s")

    @functools.partial(
        pl.kernel,
        out_type=(
            jax.ShapeDtypeStruct((N_EDGES,), _f32),
            jax.ShapeDtypeStruct((N_EDGES,), _f32),
            jax.ShapeDtypeStruct((N_EDGES,), _f32),
        ),
        mesh=mesh,
        compiler_params=pltpu.CompilerParams(needs_layout_passes=False, use_tc_tiling_on_sc=False),
        scratch_types=(
            pltpu.VMEM((C,), _i32),
            pltpu.VMEM((C,), _i32),
            pltpu.VMEM((C,), _i32),
            pltpu.VMEM((C,), _i32),
            pltpu.VMEM((C, D), _f32),
            pltpu.VMEM((C, D), _f32),
            pltpu.VMEM((C, D), _f32),
            pltpu.VMEM((C, D), _f32),
            pltpu.VMEM((C,), _f32),
            pltpu.VMEM((C,), _f32),
            pltpu.VMEM((C,), _f32),
            pltpu.VMEM((C,), _f32),
            pltpu.VMEM((C,), _f32),
            pltpu.VMEM((C,), _f32),
            pltpu.SemaphoreType.DMA,
            pltpu.SemaphoreType.DMA,
            pltpu.SemaphoreType.DMA,
            pltpu.SemaphoreType.DMA,
            pltpu.SemaphoreType.DMA,
            pltpu.SemaphoreType.DMA,
        ),
    )
    def k(zs_hbm, src_hbm, dst_hbm, xy_hbm, xx_hbm, yy_hbm,
          sidx0, sidx1, didx0, didx1, srows0, srows1, drows0, drows1,
          oxy0, oxy1, oxx0, oxx1, oyy0, oyy1,
          sem_i0, sem_i1, sem_g0, sem_g1, sem_o0, sem_o1):
        wid = lax.axis_index("s") * NC + lax.axis_index("c")
        base = wid * EW
        sidx = (sidx0, sidx1)
        didx = (didx0, didx1)
        srows = (srows0, srows1)
        drows = (drows0, drows1)
        oxy = (oxy0, oxy1)
        oxx = (oxx0, oxx1)
        oyy = (oyy0, oyy1)
        sem_i = (sem_i0, sem_i1)
        sem_g = (sem_g0, sem_g1)
        sem_o = (sem_o0, sem_o1)

        def start_idx(gg, b):
            off = base + gg * C
            pltpu.async_copy(src_hbm.at[pl.ds(off, C)], sidx[b], sem_i[b])
            pltpu.async_copy(dst_hbm.at[pl.ds(off, C)], didx[b], sem_i[b])

        def wait_idx(gg, b):
            off = base + gg * C
            pltpu.make_async_copy(src_hbm.at[pl.ds(off, C)], sidx[b], sem_i[b]).wait()
            pltpu.make_async_copy(dst_hbm.at[pl.ds(off, C)], didx[b], sem_i[b]).wait()

        def start_gather(b):
            pltpu.async_copy(zs_hbm.at[sidx[b]], srows[b], sem_g[b])
            pltpu.async_copy(zs_hbm.at[didx[b]], drows[b], sem_g[b])

        def wait_gather(b):
            pltpu.make_async_copy(zs_hbm.at[sidx[b]], srows[b], sem_g[b]).wait()
            pltpu.make_async_copy(zs_hbm.at[didx[b]], drows[b], sem_g[b]).wait()

        def start_out(gg, b):
            off = base + gg * C
            pltpu.async_copy(oxy[b], xy_hbm.at[pl.ds(off, C)], sem_o[b])
            pltpu.async_copy(oxx[b], xx_hbm.at[pl.ds(off, C)], sem_o[b])
            pltpu.async_copy(oyy[b], yy_hbm.at[pl.ds(off, C)], sem_o[b])

        def wait_out(gg, b):
            off = base + gg * C
            pltpu.make_async_copy(oxy[b], xy_hbm.at[pl.ds(off, C)], sem_o[b]).wait()
            pltpu.make_async_copy(oxx[b], xx_hbm.at[pl.ds(off, C)], sem_o[b]).wait()
            pltpu.make_async_copy(oyy[b], yy_hbm.at[pl.ds(off, C)], sem_o[b]).wait()

        def compute_chunk(b):
            sr = srows[b]
            dr = drows[b]
            i16 = lax.iota(_i32, 16)

            def group_body(g, carry):
                e0 = g * 16
                rows = i16 + e0
                z = jnp.zeros((16,), _f32)
                xy0, xy1, xx0, xx1, yy0, yy1 = z, z, z, z, z, z
                for d in range(D):
                    col = jnp.full((16,), d, _i32)
                    a = plsc.load_gather(sr, [rows, col])
                    c = plsc.load_gather(dr, [rows, col])
                    if d % 2 == 0:
                        xy0 = xy0 + a * c
                        xx0 = xx0 + a * a
                        yy0 = yy0 + c * c
                    else:
                        xy1 = xy1 + a * c
                        xx1 = xx1 + a * a
                        yy1 = yy1 + c * c
                oxy[b][pl.ds(e0, 16)] = xy0 + xy1
                oxx[b][pl.ds(e0, 16)] = xx0 + xx1
                oyy[b][pl.ds(e0, 16)] = yy0 + yy1
                return carry

            def unrolled_body(u, carry):
                for j in range(5):
                    group_body(u * 5 + j, carry)
                return carry

            lax.fori_loop(0, GROUPS // 5, unrolled_body, 0)

        start_idx(0, 0)
        start_idx(1, 1)
        wait_idx(0, 0)
        start_gather(0)

        def loop_body(i, carry):
            for b in range(2):
                gg = 2 * i + b
                wait_gather(b)

                @pl.when(gg + 2 < NCH)
                def _():
                    start_idx(gg + 2, b)

                @pl.when(gg + 1 < NCH)
                def _():
                    wait_idx(gg + 1, 1 - b)
                    start_gather(1 - b)

                @pl.when(gg >= 2)
                def _():
                    wait_out(gg - 2, b)

                compute_chunk(b)
                start_out(gg, b)
            return carry

        lax.fori_loop(0, NCH // 2, loop_body, 0)
        wait_out(NCH - 2, 0)
        wait_out(NCH - 1, 1)

    return k(zs, src, dst)


_ROWS2D = 25000
_BR = 1000
_NBLK = _ROWS2D // _BR


def _tc_tail_body(s_ref, xy_ref, xx_ref, yy_ref, o_ref):
    r = s_ref[0]
    t = s_ref[1]
    xy = xy_ref[...]
    xx = xx_ref[...]
    yy = yy_ref[...]
    x0 = jnp.sqrt(1.0 + xx)
    y0 = jnp.sqrt(1.0 + yy)
    arg = jnp.maximum(x0 * y0 - xy, _f32(1.0 + 1e-7))
    dist = jnp.log(arg + jnp.sqrt((arg - 1.0) * (arg + 1.0)))
    o_ref[...] = 1.0 / (jnp.exp((dist - r) / t) + 1.0)


def _tc_tail(xy, xx, yy, r, t):
    params = jnp.stack([r, t]).astype(_f32)
    xy2 = xy.reshape(_ROWS2D, 128)
    xx2 = xx.reshape(_ROWS2D, 128)
    yy2 = yy.reshape(_ROWS2D, 128)
    blk = lambda: pl.BlockSpec((_BR, 128), lambda i: (i, 0))
    probs2 = pl.pallas_call(
        _tc_tail_body,
        grid=(_NBLK,),
        in_specs=[
            pl.BlockSpec(memory_space=pltpu.SMEM),
            blk(), blk(), blk(),
        ],
        out_specs=blk(),
        out_shape=jax.ShapeDtypeStruct((_ROWS2D, 128), _f32),
    )(params, xy2, xx2, yy2)
    return probs2.reshape(N_EDGES)


def kernel(z, edge_index, r, t):
    zs = z[:, 1:]
    src = edge_index[0]
    dst = edge_index[1]
    xy, xx, yy = _sc_edge_dots(zs, src, dst)
    return _tc_tail(xy, xx, yy, r, t)


# speedup vs baseline: 1.4804x; 1.4804x over previous
import functools

import jax
import jax.numpy as jnp
from jax import lax
from jax.experimental import pallas as pl
from jax.experimental.pallas import tpu as pltpu
from jax.experimental.pallas import tpu_sc as plsc

N_NODES = 100000
N_EDGES = 3200000
D = 16
NC, NS = 2, 16
NW = NC * NS
EW = N_EDGES // NW
C = 400
NCH = EW // C
GROUPS = C // 16

_f32 = jnp.float32
_i32 = jnp.int32


def _sc_edge_dots(zs, src, dst):
    mesh = plsc.VectorSubcoreMesh(core_axis_name="c", subcore_axis_name="s")

    @functools.partial(
        pl.kernel,
        out_type=(
            jax.ShapeDtypeStruct((N_EDGES,), _f32),
            jax.ShapeDtypeStruct((N_EDGES,), _f32),
            jax.ShapeDtypeStruct((N_EDGES,), _f32),
        ),
        mesh=mesh,
        compiler_params=pltpu.CompilerParams(needs_layout_passes=False, use_tc_tiling_on_sc=False),
        scratch_types=(
            pltpu.VMEM((C,), _i32),
            pltpu.VMEM((C,), _i32),
            pltpu.VMEM((C,), _i32),
            pltpu.VMEM((C,), _i32),
            pltpu.VMEM((C, D), _f32),
            pltpu.VMEM((C, D), _f32),
            pltpu.VMEM((C, D), _f32),
            pltpu.VMEM((C, D), _f32),
            pltpu.VMEM((C,), _f32),
            pltpu.VMEM((C,), _f32),
            pltpu.VMEM((C,), _f32),
            pltpu.VMEM((C,), _f32),
            pltpu.VMEM((C,), _f32),
            pltpu.VMEM((C,), _f32),
            pltpu.SemaphoreType.DMA,
            pltpu.SemaphoreType.DMA,
            pltpu.SemaphoreType.DMA,
            pltpu.SemaphoreType.DMA,
            pltpu.SemaphoreType.DMA,
            pltpu.SemaphoreType.DMA,
        ),
    )
    def k(zs_hbm, src_hbm, dst_hbm, xy_hbm, xx_hbm, yy_hbm,
          sidx0, sidx1, didx0, didx1, srows0, srows1, drows0, drows1,
          oxy0, oxy1, oxx0, oxx1, oyy0, oyy1,
          sem_i0, sem_i1, sem_g0, sem_g1, sem_o0, sem_o1):
        wid = lax.axis_index("s") * NC + lax.axis_index("c")
        base = wid * EW
        sidx = (sidx0, sidx1)
        didx = (didx0, didx1)
        srows = (srows0, srows1)
        drows = (drows0, drows1)
        oxy = (oxy0, oxy1)
        oxx = (oxx0, oxx1)
        oyy = (oyy0, oyy1)
        sem_i = (sem_i0, sem_i1)
        sem_g = (sem_g0, sem_g1)
        sem_o = (sem_o0, sem_o1)

        def start_idx(gg, b):
            off = base + gg * C
            pltpu.async_copy(src_hbm.at[pl.ds(off, C)], sidx[b], sem_i[b])
            pltpu.async_copy(dst_hbm.at[pl.ds(off, C)], didx[b], sem_i[b])

        def wait_idx(gg, b):
            off = base + gg * C
            pltpu.make_async_copy(src_hbm.at[pl.ds(off, C)], sidx[b], sem_i[b]).wait()
            pltpu.make_async_copy(dst_hbm.at[pl.ds(off, C)], didx[b], sem_i[b]).wait()

        def start_gather(b):
            pltpu.async_copy(zs_hbm.at[sidx[b]], srows[b], sem_g[b])
            pltpu.async_copy(zs_hbm.at[didx[b]], drows[b], sem_g[b])

        def wait_gather(b):
            pltpu.make_async_copy(zs_hbm.at[sidx[b]], srows[b], sem_g[b]).wait()
            pltpu.make_async_copy(zs_hbm.at[didx[b]], drows[b], sem_g[b]).wait()

        def start_out(gg, b):
            off = base + gg * C
            pltpu.async_copy(oxy[b], xy_hbm.at[pl.ds(off, C)], sem_o[b])
            pltpu.async_copy(oxx[b], xx_hbm.at[pl.ds(off, C)], sem_o[b])
            pltpu.async_copy(oyy[b], yy_hbm.at[pl.ds(off, C)], sem_o[b])

        def wait_out(gg, b):
            off = base + gg * C
            pltpu.make_async_copy(oxy[b], xy_hbm.at[pl.ds(off, C)], sem_o[b]).wait()
            pltpu.make_async_copy(oxx[b], xx_hbm.at[pl.ds(off, C)], sem_o[b]).wait()
            pltpu.make_async_copy(oyy[b], yy_hbm.at[pl.ds(off, C)], sem_o[b]).wait()

        def compute_chunk(b):
            sr = srows[b]
            dr = drows[b]
            i16 = lax.iota(_i32, 16)

            def group_body(g, carry):
                e0 = g * 16
                rows = i16 + e0
                z = jnp.zeros((16,), _f32)
                xy0, xy1, xx0, xx1, yy0, yy1 = z, z, z, z, z, z
                for d in range(D):
                    col = (i16 + d) & (D - 1)
                    a = plsc.load_gather(sr, [rows, col])
                    c = plsc.load_gather(dr, [rows, col])
                    if d % 2 == 0:
                        xy0 = xy0 + a * c
                        xx0 = xx0 + a * a
                        yy0 = yy0 + c * c
                    else:
                        xy1 = xy1 + a * c
                        xx1 = xx1 + a * a
                        yy1 = yy1 + c * c
                oxy[b][pl.ds(e0, 16)] = xy0 + xy1
                oxx[b][pl.ds(e0, 16)] = xx0 + xx1
                oyy[b][pl.ds(e0, 16)] = yy0 + yy1
                return carry

            def unrolled_body(u, carry):
                for j in range(5):
                    group_body(u * 5 + j, carry)
                return carry

            lax.fori_loop(0, GROUPS // 5, unrolled_body, 0)

        start_idx(0, 0)
        start_idx(1, 1)
        wait_idx(0, 0)
        start_gather(0)

        def loop_body(i, carry):
            for b in range(2):
                gg = 2 * i + b
                wait_gather(b)

                @pl.when(gg + 2 < NCH)
                def _():
                    start_idx(gg + 2, b)

                @pl.when(gg + 1 < NCH)
                def _():
                    wait_idx(gg + 1, 1 - b)
                    start_gather(1 - b)

                @pl.when(gg >= 2)
                def _():
                    wait_out(gg - 2, b)

                compute_chunk(b)
                start_out(gg, b)
            return carry

        lax.fori_loop(0, NCH // 2, loop_body, 0)
        wait_out(NCH - 2, 0)
        wait_out(NCH - 1, 1)

    return k(zs, src, dst)


_ROWS2D = 25000
_BR = 1000
_NBLK = _ROWS2D // _BR


def _tc_tail_body(s_ref, xy_ref, xx_ref, yy_ref, o_ref):
    r = s_ref[0]
    t = s_ref[1]
    xy = xy_ref[...]
    xx = xx_ref[...]
    yy = yy_ref[...]
    x0 = jnp.sqrt(1.0 + xx)
    y0 = jnp.sqrt(1.0 + yy)
    arg = jnp.maximum(x0 * y0 - xy, _f32(1.0 + 1e-7))
    dist = jnp.log(arg + jnp.sqrt((arg - 1.0) * (arg + 1.0)))
    o_ref[...] = 1.0 / (jnp.exp((dist - r) / t) + 1.0)


def _tc_tail(xy, xx, yy, r, t):
    params = jnp.stack([r, t]).astype(_f32)
    xy2 = xy.reshape(_ROWS2D, 128)
    xx2 = xx.reshape(_ROWS2D, 128)
    yy2 = yy.reshape(_ROWS2D, 128)
    blk = lambda: pl.BlockSpec((_BR, 128), lambda i: (i, 0))
    probs2 = pl.pallas_call(
        _tc_tail_body,
        grid=(_NBLK,),
        in_specs=[
            pl.BlockSpec(memory_space=pltpu.SMEM),
            blk(), blk(), blk(),
        ],
        out_specs=blk(),
        out_shape=jax.ShapeDtypeStruct((_ROWS2D, 128), _f32),
    )(params, xy2, xx2, yy2)
    return probs2.reshape(N_EDGES)


def kernel(z, edge_index, r, t):
    zs = z[:, 1:]
    src = edge_index[0]
    dst = edge_index[1]
    xy, xx, yy = _sc_edge_dots(zs, src, dst)
    return _tc_tail(xy, xx, yy, r, t)


# speedup vs baseline: 1.7567x; 1.1866x over previous
import functools

import jax
import jax.numpy as jnp
from jax import lax
from jax.experimental import pallas as pl
from jax.experimental.pallas import tpu as pltpu
from jax.experimental.pallas import tpu_sc as plsc

N_NODES = 100000
N_EDGES = 3200000
D = 16
NC, NS = 2, 16
NW = NC * NS
EW = N_EDGES // NW
C = 800
NCH = EW // C
GROUPS = C // 16

_f32 = jnp.float32
_i32 = jnp.int32


def _sc_edge_dots(zs, src, dst):
    mesh = plsc.VectorSubcoreMesh(core_axis_name="c", subcore_axis_name="s")

    @functools.partial(
        pl.kernel,
        out_type=(
            jax.ShapeDtypeStruct((N_EDGES,), _f32),
            jax.ShapeDtypeStruct((N_EDGES,), _f32),
            jax.ShapeDtypeStruct((N_EDGES,), _f32),
        ),
        mesh=mesh,
        compiler_params=pltpu.CompilerParams(needs_layout_passes=False, use_tc_tiling_on_sc=False),
        scratch_types=(
            pltpu.VMEM((C,), _i32),
            pltpu.VMEM((C,), _i32),
            pltpu.VMEM((C,), _i32),
            pltpu.VMEM((C,), _i32),
            pltpu.VMEM((C, D), _f32),
            pltpu.VMEM((C, D), _f32),
            pltpu.VMEM((C, D), _f32),
            pltpu.VMEM((C, D), _f32),
            pltpu.VMEM((C,), _f32),
            pltpu.VMEM((C,), _f32),
            pltpu.VMEM((C,), _f32),
            pltpu.VMEM((C,), _f32),
            pltpu.VMEM((C,), _f32),
            pltpu.VMEM((C,), _f32),
            pltpu.SemaphoreType.DMA,
            pltpu.SemaphoreType.DMA,
            pltpu.SemaphoreType.DMA,
            pltpu.SemaphoreType.DMA,
            pltpu.SemaphoreType.DMA,
            pltpu.SemaphoreType.DMA,
        ),
    )
    def k(zs_hbm, src_hbm, dst_hbm, xy_hbm, xx_hbm, yy_hbm,
          sidx0, sidx1, didx0, didx1, srows0, srows1, drows0, drows1,
          oxy0, oxy1, oxx0, oxx1, oyy0, oyy1,
          sem_i0, sem_i1, sem_g0, sem_g1, sem_o0, sem_o1):
        wid = lax.axis_index("s") * NC + lax.axis_index("c")
        base = wid * EW
        sidx = (sidx0, sidx1)
        didx = (didx0, didx1)
        srows = (srows0, srows1)
        drows = (drows0, drows1)
        oxy = (oxy0, oxy1)
        oxx = (oxx0, oxx1)
        oyy = (oyy0, oyy1)
        sem_i = (sem_i0, sem_i1)
        sem_g = (sem_g0, sem_g1)
        sem_o = (sem_o0, sem_o1)

        def start_idx(gg, b):
            off = base + gg * C
            pltpu.async_copy(src_hbm.at[pl.ds(off, C)], sidx[b], sem_i[b])
            pltpu.async_copy(dst_hbm.at[pl.ds(off, C)], didx[b], sem_i[b])

        def wait_idx(gg, b):
            off = base + gg * C
            pltpu.make_async_copy(src_hbm.at[pl.ds(off, C)], sidx[b], sem_i[b]).wait()
            pltpu.make_async_copy(dst_hbm.at[pl.ds(off, C)], didx[b], sem_i[b]).wait()

        def start_gather(b):
            pltpu.async_copy(zs_hbm.at[sidx[b]], srows[b], sem_g[b])
            pltpu.async_copy(zs_hbm.at[didx[b]], drows[b], sem_g[b])

        def wait_gather(b):
            pltpu.make_async_copy(zs_hbm.at[sidx[b]], srows[b], sem_g[b]).wait()
            pltpu.make_async_copy(zs_hbm.at[didx[b]], drows[b], sem_g[b]).wait()

        def start_out(gg, b):
            off = base + gg * C
            pltpu.async_copy(oxy[b], xy_hbm.at[pl.ds(off, C)], sem_o[b])
            pltpu.async_copy(oxx[b], xx_hbm.at[pl.ds(off, C)], sem_o[b])
            pltpu.async_copy(oyy[b], yy_hbm.at[pl.ds(off, C)], sem_o[b])

        def wait_out(gg, b):
            off = base + gg * C
            pltpu.make_async_copy(oxy[b], xy_hbm.at[pl.ds(off, C)], sem_o[b]).wait()
            pltpu.make_async_copy(oxx[b], xx_hbm.at[pl.ds(off, C)], sem_o[b]).wait()
            pltpu.make_async_copy(oyy[b], yy_hbm.at[pl.ds(off, C)], sem_o[b]).wait()

        def compute_chunk(b):
            sr = srows[b]
            dr = drows[b]
            i16 = lax.iota(_i32, 16)

            def group_body(g, carry):
                e0 = g * 16
                rows = i16 + e0
                z = jnp.zeros((16,), _f32)
                xy0, xy1, xx0, xx1, yy0, yy1 = z, z, z, z, z, z
                for d in range(D):
                    col = (i16 + d) & (D - 1)
                    a = plsc.load_gather(sr, [rows, col])
                    c = plsc.load_gather(dr, [rows, col])
                    if d % 2 == 0:
                        xy0 = xy0 + a * c
                        xx0 = xx0 + a * a
                        yy0 = yy0 + c * c
                    else:
                        xy1 = xy1 + a * c
                        xx1 = xx1 + a * a
                        yy1 = yy1 + c * c
                oxy[b][pl.ds(e0, 16)] = xy0 + xy1
                oxx[b][pl.ds(e0, 16)] = xx0 + xx1
                oyy[b][pl.ds(e0, 16)] = yy0 + yy1
                return carry

            def unrolled_body(u, carry):
                for j in range(5):
                    group_body(u * 5 + j, carry)
                return carry

            lax.fori_loop(0, GROUPS // 5, unrolled_body, 0)

        start_idx(0, 0)
        start_idx(1, 1)
        wait_idx(0, 0)
        start_gather(0)

        def iteration(gg, b):
            wait_gather(b)

            @pl.when(gg + 2 < NCH)
            def _():
                start_idx(gg + 2, b)

            @pl.when(gg + 1 < NCH)
            def _():
                wait_idx(gg + 1, 1 - b)
                start_gather(1 - b)

            @pl.when(gg >= 2)
            def _():
                wait_out(gg - 2, b)

            compute_chunk(b)
            start_out(gg, b)

        def loop_body(i, carry):
            for b in range(2):
                iteration(2 * i + b, b)
            return carry

        lax.fori_loop(0, NCH // 2, loop_body, 0)
        if NCH % 2:
            iteration(NCH - 1, (NCH - 1) % 2)
        wait_out(NCH - 2, (NCH - 2) % 2)
        wait_out(NCH - 1, (NCH - 1) % 2)

    return k(zs, src, dst)


_ROWS2D = 25000
_BR = 1000
_NBLK = _ROWS2D // _BR


def _tc_tail_body(s_ref, xy_ref, xx_ref, yy_ref, o_ref):
    r = s_ref[0]
    t = s_ref[1]
    xy = xy_ref[...]
    xx = xx_ref[...]
    yy = yy_ref[...]
    x0 = jnp.sqrt(1.0 + xx)
    y0 = jnp.sqrt(1.0 + yy)
    arg = jnp.maximum(x0 * y0 - xy, _f32(1.0 + 1e-7))
    dist = jnp.log(arg + jnp.sqrt((arg - 1.0) * (arg + 1.0)))
    o_ref[...] = 1.0 / (jnp.exp((dist - r) / t) + 1.0)


def _tc_tail(xy, xx, yy, r, t):
    params = jnp.stack([r, t]).astype(_f32)
    xy2 = xy.reshape(_ROWS2D, 128)
    xx2 = xx.reshape(_ROWS2D, 128)
    yy2 = yy.reshape(_ROWS2D, 128)
    blk = lambda: pl.BlockSpec((_BR, 128), lambda i: (i, 0))
    probs2 = pl.pallas_call(
        _tc_tail_body,
        grid=(_NBLK,),
        in_specs=[
            pl.BlockSpec(memory_space=pltpu.SMEM),
            blk(), blk(), blk(),
        ],
        out_specs=blk(),
        out_shape=jax.ShapeDtypeStruct((_ROWS2D, 128), _f32),
    )(params, xy2, xx2, yy2)
    return probs2.reshape(N_EDGES)


def kernel(z, edge_index, r, t):
    zs = z[:, 1:]
    src = edge_index[0]
    dst = edge_index[1]
    xy, xx, yy = _sc_edge_dots(zs, src, dst)
    return _tc_tail(xy, xx, yy, r, t)


# speedup vs baseline: 1.8534x; 1.0550x over previous
import functools

import jax
import jax.numpy as jnp
from jax import lax
from jax.experimental import pallas as pl
from jax.experimental.pallas import tpu as pltpu
from jax.experimental.pallas import tpu_sc as plsc

N_NODES = 100000
N_EDGES = 3200000
D = 16
NC, NS = 2, 16
NW = NC * NS
EW = N_EDGES // NW
C = 800
NCH = EW // C
GROUPS = C // 16

_f32 = jnp.float32
_i32 = jnp.int32


def _sc_edge_dots(zs, src, dst):
    mesh = plsc.VectorSubcoreMesh(core_axis_name="c", subcore_axis_name="s")

    @functools.partial(
        pl.kernel,
        out_type=jax.ShapeDtypeStruct((N_EDGES,), _f32),
        mesh=mesh,
        compiler_params=pltpu.CompilerParams(needs_layout_passes=False, use_tc_tiling_on_sc=False),
        scratch_types=(
            pltpu.VMEM((C,), _i32),
            pltpu.VMEM((C,), _i32),
            pltpu.VMEM((C,), _i32),
            pltpu.VMEM((C,), _i32),
            pltpu.VMEM((C, D), _f32),
            pltpu.VMEM((C, D), _f32),
            pltpu.VMEM((C, D), _f32),
            pltpu.VMEM((C, D), _f32),
            pltpu.VMEM((C,), _f32),
            pltpu.VMEM((C,), _f32),
            pltpu.SemaphoreType.DMA,
            pltpu.SemaphoreType.DMA,
            pltpu.SemaphoreType.DMA,
            pltpu.SemaphoreType.DMA,
            pltpu.SemaphoreType.DMA,
            pltpu.SemaphoreType.DMA,
        ),
    )
    def k(zs_hbm, src_hbm, dst_hbm, arg_hbm,
          sidx0, sidx1, didx0, didx1, srows0, srows1, drows0, drows1,
          oarg0, oarg1,
          sem_i0, sem_i1, sem_g0, sem_g1, sem_o0, sem_o1):
        wid = lax.axis_index("s") * NC + lax.axis_index("c")
        base = wid * EW
        sidx = (sidx0, sidx1)
        didx = (didx0, didx1)
        srows = (srows0, srows1)
        drows = (drows0, drows1)
        oarg = (oarg0, oarg1)
        sem_i = (sem_i0, sem_i1)
        sem_g = (sem_g0, sem_g1)
        sem_o = (sem_o0, sem_o1)

        def start_idx(gg, b):
            off = base + gg * C
            pltpu.async_copy(src_hbm.at[pl.ds(off, C)], sidx[b], sem_i[b])
            pltpu.async_copy(dst_hbm.at[pl.ds(off, C)], didx[b], sem_i[b])

        def wait_idx(gg, b):
            off = base + gg * C
            pltpu.make_async_copy(src_hbm.at[pl.ds(off, C)], sidx[b], sem_i[b]).wait()
            pltpu.make_async_copy(dst_hbm.at[pl.ds(off, C)], didx[b], sem_i[b]).wait()

        def start_gather(b):
            pltpu.async_copy(zs_hbm.at[sidx[b]], srows[b], sem_g[b])
            pltpu.async_copy(zs_hbm.at[didx[b]], drows[b], sem_g[b])

        def wait_gather(b):
            pltpu.make_async_copy(zs_hbm.at[sidx[b]], srows[b], sem_g[b]).wait()
            pltpu.make_async_copy(zs_hbm.at[didx[b]], drows[b], sem_g[b]).wait()

        def start_out(gg, b):
            off = base + gg * C
            pltpu.async_copy(oarg[b], arg_hbm.at[pl.ds(off, C)], sem_o[b])

        def wait_out(gg, b):
            off = base + gg * C
            pltpu.make_async_copy(oarg[b], arg_hbm.at[pl.ds(off, C)], sem_o[b]).wait()

        def _sqrt1p(v):
            x = 1.0 + v
            y = plsc.bitcast(0x5F3759DF - (plsc.bitcast(x, _i32) >> 1), _f32)
            hx = 0.5 * x
            y = y * (1.5 - hx * y * y)
            y = y * (1.5 - hx * y * y)
            y = y * (1.5 - hx * y * y)
            return x * y

        def compute_chunk(b):
            sr = srows[b]
            dr = drows[b]
            i16 = lax.iota(_i32, 16)

            def group_body(g, carry):
                e0 = g * 16
                rows = i16 + e0
                z = jnp.zeros((16,), _f32)
                xy0, xy1, xx0, xx1, yy0, yy1 = z, z, z, z, z, z
                for d in range(D):
                    col = (i16 + d) & (D - 1)
                    a = plsc.load_gather(sr, [rows, col])
                    c = plsc.load_gather(dr, [rows, col])
                    if d % 2 == 0:
                        xy0 = xy0 + a * c
                        xx0 = xx0 + a * a
                        yy0 = yy0 + c * c
                    else:
                        xy1 = xy1 + a * c
                        xx1 = xx1 + a * a
                        yy1 = yy1 + c * c
                x0 = _sqrt1p(xx0 + xx1)
                y0 = _sqrt1p(yy0 + yy1)
                oarg[b][pl.ds(e0, 16)] = x0 * y0 - (xy0 + xy1)
                return carry

            def unrolled_body(u, carry):
                for j in range(5):
                    group_body(u * 5 + j, carry)
                return carry

            lax.fori_loop(0, GROUPS // 5, unrolled_body, 0)

        start_idx(0, 0)
        start_idx(1, 1)
        wait_idx(0, 0)
        start_gather(0)

        def iteration(gg, b):
            wait_gather(b)

            @pl.when(gg + 2 < NCH)
            def _():
                start_idx(gg + 2, b)

            @pl.when(gg + 1 < NCH)
            def _():
                wait_idx(gg + 1, 1 - b)
                start_gather(1 - b)

            @pl.when(gg >= 2)
            def _():
                wait_out(gg - 2, b)

            compute_chunk(b)
            start_out(gg, b)

        def loop_body(i, carry):
            for b in range(2):
                iteration(2 * i + b, b)
            return carry

        lax.fori_loop(0, NCH // 2, loop_body, 0)
        if NCH % 2:
            iteration(NCH - 1, (NCH - 1) % 2)
        wait_out(NCH - 2, (NCH - 2) % 2)
        wait_out(NCH - 1, (NCH - 1) % 2)

    return k(zs, src, dst)


_ROWS2D = 25000
_BR = 1000
_NBLK = _ROWS2D // _BR


def _tc_tail_body(s_ref, arg_ref, o_ref):
    r = s_ref[0]
    t = s_ref[1]
    arg = jnp.maximum(arg_ref[...], _f32(1.0 + 1e-7))
    dist = jnp.log(arg + jnp.sqrt((arg - 1.0) * (arg + 1.0)))
    o_ref[...] = 1.0 / (jnp.exp((dist - r) / t) + 1.0)


def _tc_tail(arg, r, t):
    params = jnp.stack([r, t]).astype(_f32)
    arg2 = arg.reshape(_ROWS2D, 128)
    blk = lambda: pl.BlockSpec((_BR, 128), lambda i: (i, 0))
    probs2 = pl.pallas_call(
        _tc_tail_body,
        grid=(_NBLK,),
        in_specs=[pl.BlockSpec(memory_space=pltpu.SMEM), blk()],
        out_specs=blk(),
        out_shape=jax.ShapeDtypeStruct((_ROWS2D, 128), _f32),
    )(params, arg2)
    return probs2.reshape(N_EDGES)


def kernel(z, edge_index, r, t):
    zs = z[:, 1:]
    src = edge_index[0]
    dst = edge_index[1]
    arg = _sc_edge_dots(zs, src, dst)
    return _tc_tail(arg, r, t)


# speedup vs baseline: 1.9649x; 1.0602x over previous
import functools

import jax
import jax.numpy as jnp
from jax import lax
from jax.experimental import pallas as pl
from jax.experimental.pallas import tpu as pltpu
from jax.experimental.pallas import tpu_sc as plsc

N_NODES = 100000
N_EDGES = 3200000
D = 16
NC, NS = 2, 16
NW = NC * NS
EW = N_EDGES // NW
C = 800
NCH = EW // C
GROUPS = C // 16

_f32 = jnp.float32
_i32 = jnp.int32


def _sc_edge_dots(zs, src, dst):
    mesh = plsc.VectorSubcoreMesh(core_axis_name="c", subcore_axis_name="s")

    @functools.partial(
        pl.kernel,
        out_type=jax.ShapeDtypeStruct((N_EDGES,), _f32),
        mesh=mesh,
        compiler_params=pltpu.CompilerParams(needs_layout_passes=False, use_tc_tiling_on_sc=False),
        scratch_types=(
            [pltpu.VMEM((C,), _i32)] * 6
            + [pltpu.VMEM((C, D), _f32)] * 6
            + [pltpu.VMEM((C,), _f32)] * 3
            + [pltpu.SemaphoreType.DMA] * 9
        ),
    )
    def k(zs_hbm, src_hbm, dst_hbm, arg_hbm,
          sidx0, sidx1, sidx2, didx0, didx1, didx2,
          srows0, srows1, srows2, drows0, drows1, drows2,
          oarg0, oarg1, oarg2,
          sem_i0, sem_i1, sem_i2, sem_g0, sem_g1, sem_g2,
          sem_o0, sem_o1, sem_o2):
        wid = lax.axis_index("s") * NC + lax.axis_index("c")
        base = wid * EW
        sidx = (sidx0, sidx1, sidx2)
        didx = (didx0, didx1, didx2)
        srows = (srows0, srows1, srows2)
        drows = (drows0, drows1, drows2)
        oarg = (oarg0, oarg1, oarg2)
        sem_i = (sem_i0, sem_i1, sem_i2)
        sem_g = (sem_g0, sem_g1, sem_g2)
        sem_o = (sem_o0, sem_o1, sem_o2)

        def start_idx(gg, b):
            off = base + gg * C
            pltpu.async_copy(src_hbm.at[pl.ds(off, C)], sidx[b], sem_i[b])
            pltpu.async_copy(dst_hbm.at[pl.ds(off, C)], didx[b], sem_i[b])

        def wait_idx(gg, b):
            off = base + gg * C
            pltpu.make_async_copy(src_hbm.at[pl.ds(off, C)], sidx[b], sem_i[b]).wait()
            pltpu.make_async_copy(dst_hbm.at[pl.ds(off, C)], didx[b], sem_i[b]).wait()

        def start_gather(b):
            pltpu.async_copy(zs_hbm.at[sidx[b]], srows[b], sem_g[b])
            pltpu.async_copy(zs_hbm.at[didx[b]], drows[b], sem_g[b])

        def wait_gather(b):
            pltpu.make_async_copy(zs_hbm.at[sidx[b]], srows[b], sem_g[b]).wait()
            pltpu.make_async_copy(zs_hbm.at[didx[b]], drows[b], sem_g[b]).wait()

        def start_out(gg, b):
            off = base + gg * C
            pltpu.async_copy(oarg[b], arg_hbm.at[pl.ds(off, C)], sem_o[b])

        def wait_out(gg, b):
            off = base + gg * C
            pltpu.make_async_copy(oarg[b], arg_hbm.at[pl.ds(off, C)], sem_o[b]).wait()

        def _sqrt1p(v):
            x = 1.0 + v
            y = plsc.bitcast(0x5F3759DF - (plsc.bitcast(x, _i32) >> 1), _f32)
            hx = 0.5 * x
            y = y * (1.5 - hx * y * y)
            y = y * (1.5 - hx * y * y)
            y = y * (1.5 - hx * y * y)
            return x * y

        def compute_chunk(b):
            sr = srows[b]
            dr = drows[b]
            i16 = lax.iota(_i32, 16)

            def group_body(g, carry):
                e0 = g * 16
                rows = i16 + e0
                z = jnp.zeros((16,), _f32)
                xy0, xy1, xx0, xx1, yy0, yy1 = z, z, z, z, z, z
                for d in range(D):
                    col = (i16 + d) & (D - 1)
                    a = plsc.load_gather(sr, [rows, col])
                    c = plsc.load_gather(dr, [rows, col])
                    if d % 2 == 0:
                        xy0 = xy0 + a * c
                        xx0 = xx0 + a * a
                        yy0 = yy0 + c * c
                    else:
                        xy1 = xy1 + a * c
                        xx1 = xx1 + a * a
                        yy1 = yy1 + c * c
                x0 = _sqrt1p(xx0 + xx1)
                y0 = _sqrt1p(yy0 + yy1)
                oarg[b][pl.ds(e0, 16)] = x0 * y0 - (xy0 + xy1)
                return carry

            def unrolled_body(u, carry):
                for j in range(5):
                    group_body(u * 5 + j, carry)
                return carry

            lax.fori_loop(0, GROUPS // 5, unrolled_body, 0)

        start_idx(0, 0)
        start_idx(1, 1)
        start_idx(2, 2)
        wait_idx(0, 0)
        start_gather(0)
        wait_idx(1, 1)
        start_gather(1)

        def iteration(gg, b):
            wait_gather(b)

            @pl.when(gg + 3 < NCH)
            def _():
                start_idx(gg + 3, b)

            @pl.when(gg + 2 < NCH)
            def _():
                wait_idx(gg + 2, (b + 2) % 3)
                start_gather((b + 2) % 3)

            @pl.when(gg >= 3)
            def _():
                wait_out(gg - 3, b)

            compute_chunk(b)
            start_out(gg, b)

        def loop_body(i, carry):
            for b in range(3):
                iteration(3 * i + b, b)
            return carry

        lax.fori_loop(0, NCH // 3, loop_body, 0)
        for gg in range(NCH - NCH % 3, NCH):
            iteration(gg, gg % 3)
        for gg in range(NCH - 3, NCH):
            wait_out(gg, gg % 3)

    return k(zs, src, dst)


_ROWS2D = 25000
_BR = 1000
_NBLK = _ROWS2D // _BR


def _tc_tail_body(s_ref, arg_ref, o_ref):
    r = s_ref[0]
    t = s_ref[1]
    arg = jnp.maximum(arg_ref[...], _f32(1.0 + 1e-7))
    dist = jnp.log(arg + jnp.sqrt((arg - 1.0) * (arg + 1.0)))
    o_ref[...] = 1.0 / (jnp.exp((dist - r) / t) + 1.0)


def _tc_tail(arg, r, t):
    params = jnp.stack([r, t]).astype(_f32)
    arg2 = arg.reshape(_ROWS2D, 128)
    blk = lambda: pl.BlockSpec((_BR, 128), lambda i: (i, 0))
    probs2 = pl.pallas_call(
        _tc_tail_body,
        grid=(_NBLK,),
        in_specs=[pl.BlockSpec(memory_space=pltpu.SMEM), blk()],
        out_specs=blk(),
        out_shape=jax.ShapeDtypeStruct((_ROWS2D, 128), _f32),
    )(params, arg2)
    return probs2.reshape(N_EDGES)


def kernel(z, edge_index, r, t):
    zs = z[:, 1:]
    src = edge_index[0]
    dst = edge_index[1]
    arg = _sc_edge_dots(zs, src, dst)
    return _tc_tail(arg, r, t)
